# baseline (speedup 1.0000x reference)
import functools

import jax
import jax.numpy as jnp
from jax import lax
from jax.experimental import pallas as pl
from jax.experimental.pallas import tpu as pltpu
from jax.experimental.pallas import tpu_sc as plsc

_NC = 2
_NS = 16
_NW = _NC * _NS
_CB = 16


def _sc_call(questions, table):
    b, l = questions.shape
    d = table.shape[1]
    nk = d // 16
    rows_per_w = b // _NW
    nchunk = rows_per_w // _CB
    mesh = plsc.VectorSubcoreMesh(core_axis_name="c", subcore_axis_name="s")

    @functools.partial(
        pl.kernel,
        out_type=jax.ShapeDtypeStruct((b, d), jnp.float32),
        mesh=mesh,
        scratch_types=[
            pltpu.VMEM((2, _CB, l), jnp.int32),
            pltpu.VMEM((2, _CB * l, d), jnp.bfloat16),
            pltpu.VMEM((2, _CB, d), jnp.float32),
            pltpu.SemaphoreType.DMA,
            pltpu.SemaphoreType.DMA,
        ],
        compiler_params=pltpu.CompilerParams(
            use_tc_tiling_on_sc=False, needs_layout_passes=False
        ),
    )
    def k(q_hbm, t_hbm, out_hbm, idx_v, rows_v, out_v, sem0, sem1):
        wid = lax.axis_index("s") * _NC + lax.axis_index("c")
        base_row = wid * rows_per_w
        sems = [sem0, sem1]

        lane2 = lax.iota(jnp.int32, 16) * 2
        beta = [
            ((lane2 + off).astype(jnp.float32) - 31.0) * (1.0 / 800.0)
            for off in (0, 1, 32, 33)
        ]

        def fire(ci, buf):
            row0 = base_row + ci * _CB
            pltpu.sync_copy(q_hbm.at[pl.ds(row0, _CB)], idx_v.at[buf])
            for j in range(_CB):
                pltpu.async_copy(
                    t_hbm.at[idx_v.at[buf].at[j]],
                    rows_v.at[buf].at[pl.ds(j * l, l)],
                    sems[buf],
                )

        def drain(buf):
            pltpu.make_async_copy(
                t_hbm.at[pl.ds(0, _CB * l)], rows_v.at[buf], sems[buf]
            ).wait()

        def compute(ci, buf):
            rows = rows_v.at[buf]
            row0 = base_row + ci * _CB

            def row_body(r, carry):
                base = r * l
                acc0 = [None] * nk
                acc1 = [None] * nk
                for li in range(l):
                    alpha = float(li - 24)
                    h0 = rows[base + li, pl.ds(0, 32)]
                    h1 = rows[base + li, pl.ds(32, 32)]
                    va, vb = plsc.unpack(h0, format=plsc.PackFormat.INTERLEAVED)
                    vc, vd2 = plsc.unpack(h1, format=plsc.PackFormat.INTERLEAVED)
                    for kk, v in enumerate((va, vb, vc, vd2)):
                        if li == 0:
                            acc0[kk] = v
                            acc1[kk] = alpha * v
                        else:
                            acc0[kk] = acc0[kk] + v
                            if alpha == 1.0:
                                acc1[kk] = acc1[kk] + v
                            elif alpha != 0.0:
                                acc1[kk] = acc1[kk] + alpha * v
                for kk in range(nk):
                    out_v[buf, r, pl.ds(16 * kk, 16)] = (
                        acc0[kk] + beta[kk] * acc1[kk]
                    )
                return carry

            lax.fori_loop(0, _CB, row_body, 0)
            pltpu.sync_copy(out_v.at[buf], out_hbm.at[pl.ds(row0, _CB)])

        fire(0, 0)

        def pair_body(p, carry):
            ci0 = p * 2
            for bb in range(2):
                ci = ci0 + bb
                nxt = ci + 1

                @pl.when(nxt < nchunk)
                def _():
                    fire(nxt, 1 - bb)

                drain(bb)
                compute(ci, bb)
            return carry

        lax.fori_loop(0, nchunk // 2, pair_body, 0)

    return k(questions, table)


def kernel(questions, table):
    q = questions.astype(jnp.int32)
    out_perm = _sc_call(q, table.astype(jnp.bfloat16))
    inv = [0] * 64
    for c in range(16):
        inv[2 * c] = c
        inv[2 * c + 1] = 16 + c
        inv[32 + 2 * c] = 32 + c
        inv[33 + 2 * c] = 48 + c
    return jnp.take(out_perm, jnp.array(inv, jnp.int32), axis=1)


# speedup vs baseline: 1.3332x; 1.3332x over previous
import functools

import jax
import jax.numpy as jnp
from jax import lax
from jax.experimental import pallas as pl
from jax.experimental.pallas import tpu as pltpu
from jax.experimental.pallas import tpu_sc as plsc

_NC = 2
_NS = 16
_NW = _NC * _NS
_CB = 16


def _sc_call(questions, table):
    b, l = questions.shape
    d = table.shape[1]
    nk = d // 16
    rows_per_w = b // _NW
    nchunk = rows_per_w // _CB
    mesh = plsc.VectorSubcoreMesh(core_axis_name="c", subcore_axis_name="s")

    @functools.partial(
        pl.kernel,
        out_type=jax.ShapeDtypeStruct((b, d), jnp.float32),
        mesh=mesh,
        scratch_types=[
            pltpu.VMEM((2, _CB, l), jnp.int32),
            pltpu.VMEM((2, _CB * l, d), jnp.float32),
            pltpu.VMEM((2, _CB, d), jnp.float32),
            pltpu.SemaphoreType.DMA,
            pltpu.SemaphoreType.DMA,
        ],
        compiler_params=pltpu.CompilerParams(use_tc_tiling_on_sc=False),
    )
    def k(q_hbm, t_hbm, out_hbm, idx_v, rows_v, out_v, sem0, sem1):
        wid = lax.axis_index("s") * _NC + lax.axis_index("c")
        base_row = wid * rows_per_w
        sems = [sem0, sem1]

        beta = [
            (lax.iota(jnp.int32, 16).astype(jnp.float32) + (16.0 * kk - 31.0))
            * (1.0 / 800.0)
            for kk in range(nk)
        ]

        def fire(ci, buf):
            row0 = base_row + ci * _CB
            pltpu.sync_copy(q_hbm.at[pl.ds(row0, _CB)], idx_v.at[buf])
            for j in range(_CB):
                pltpu.async_copy(
                    t_hbm.at[idx_v.at[buf].at[j]],
                    rows_v.at[buf].at[pl.ds(j * l, l)],
                    sems[buf],
                )

        def drain(buf):
            pltpu.make_async_copy(
                t_hbm.at[pl.ds(0, _CB * l)], rows_v.at[buf], sems[buf]
            ).wait()

        def compute(ci, buf):
            rows = rows_v.at[buf]
            row0 = base_row + ci * _CB

            def row_body(r, carry):
                base = r * l
                acc0 = [None] * nk
                acc1 = [None] * nk
                for li in range(l):
                    alpha = float(li - 24)
                    for kk in range(nk):
                        v = rows[base + li, pl.ds(16 * kk, 16)]
                        if li == 0:
                            acc0[kk] = v
                            acc1[kk] = alpha * v
                        else:
                            acc0[kk] = acc0[kk] + v
                            if alpha == 1.0:
                                acc1[kk] = acc1[kk] + v
                            elif alpha != 0.0:
                                acc1[kk] = acc1[kk] + alpha * v
                for kk in range(nk):
                    out_v[buf, r, pl.ds(16 * kk, 16)] = (
                        acc0[kk] + beta[kk] * acc1[kk]
                    )
                return carry

            lax.fori_loop(0, _CB, row_body, 0)
            pltpu.sync_copy(out_v.at[buf], out_hbm.at[pl.ds(row0, _CB)])

        fire(0, 0)

        def pair_body(p, carry):
            ci0 = p * 2
            for bb in range(2):
                ci = ci0 + bb
                nxt = ci + 1

                @pl.when(nxt < nchunk)
                def _():
                    fire(nxt, 1 - bb)

                drain(bb)
                compute(ci, bb)
            return carry

        lax.fori_loop(0, nchunk // 2, pair_body, 0)

    return k(questions, table)


def kernel(questions, table):
    q = questions.astype(jnp.int32)
    return _sc_call(q, table)
